# baseline (speedup 1.0000x reference)
import functools

import jax
import jax.numpy as jnp
from jax.experimental import pallas as pl
from jax.experimental.pallas import tpu as pltpu


def _fused_body(adj_hbm, xa0_ref, w_ref, out_ref, h_s, buf0, buf1,
                sem0, sem1, *, feat, cm, nchunks):
    l = pl.program_id(0)
    ws = w_ref[0, :feat]
    wn = w_ref[0, feat:2 * feat]
    b = w_ref[0, 2 * feat:2 * feat + 1]

    def copy(c, buf, sem):
        return pltpu.make_async_copy(
            adj_hbm.at[pl.ds(c * cm, cm), :], buf, sem)

    copy(0, buf0, sem0).start()
    if nchunks > 1:
        copy(1, buf1, sem1).start()

    def epilogue(prod, xs):
        s = prod[:, :feat]
        deg = jnp.clip(prod[:, feat:feat + 1], 1e-6, None)
        neigh = s / deg
        return (jnp.dot(xs, ws, preferred_element_type=jnp.float32,
                        precision=jax.lax.Precision.HIGHEST)
                + jnp.dot(neigh, wn, preferred_element_type=jnp.float32,
                          precision=jax.lax.Precision.HIGHEST)
                + b)

    def chunk(c, buf, sem):
        copy(c, buf, sem).wait()
        a = buf[...].astype(jnp.bfloat16)
        base = pl.multiple_of(c * cm, cm)

        @pl.when(l == 0)
        def _layer1():
            prod = jnp.dot(a, xa0_ref[...],
                           preferred_element_type=jnp.float32)
            xs = xa0_ref[pl.ds(base, cm), :feat].astype(jnp.float32)
            h = jnp.maximum(epilogue(prod, xs), 0.0)
            h_s[pl.ds(base, cm), :feat] = h.astype(jnp.bfloat16)
            h_s[pl.ds(base, cm), feat:feat + 1] = jnp.ones(
                (cm, 1), jnp.bfloat16)

        @pl.when(l == 1)
        def _layer2():
            prod = jnp.dot(a, h_s[...], preferred_element_type=jnp.float32)
            xs2 = h_s[pl.ds(base, cm), :feat].astype(jnp.float32)
            out_ref[pl.ds(base, cm), :] = epilogue(prod, xs2)

        @pl.when(c + 2 < nchunks)
        def _prefetch():
            copy(c + 2, buf, sem).start()

    def pair(p, carry):
        c0 = 2 * p
        chunk(c0, buf0, sem0)

        @pl.when(c0 + 1 < nchunks)
        def _odd():
            chunk(c0 + 1, buf1, sem1)

        return carry

    jax.lax.fori_loop(0, (nchunks + 1) // 2, pair, 0)


def _pick_cm(n):
    for c in (400, 256, 200, 128, 80, 64, 40, 32, 16, 8):
        if n % c == 0:
            return c
    return n


def kernel(fts, adj, W1, b1, W2, b2):
    n, feat = fts.shape
    cm = _pick_cm(n)
    xa0 = jnp.concatenate(
        [fts.astype(jnp.bfloat16), jnp.ones((n, 1), jnp.bfloat16)], axis=1)
    wpack = jnp.stack([
        jnp.concatenate([W1[:feat], W1[feat:], b1.reshape(1, feat)], axis=0),
        jnp.concatenate([W2[:feat], W2[feat:], b2.reshape(1, feat)], axis=0),
    ])
    body = functools.partial(_fused_body, feat=feat, cm=cm, nchunks=n // cm)
    return pl.pallas_call(
        body,
        grid=(2,),
        in_specs=[
            pl.BlockSpec(memory_space=pl.ANY),
            pl.BlockSpec((n, feat + 1), lambda l: (0, 0)),
            pl.BlockSpec((1, 2 * feat + 1, feat), lambda l: (l, 0, 0)),
        ],
        out_specs=pl.BlockSpec((n, feat), lambda l: (0, 0)),
        out_shape=jax.ShapeDtypeStruct((n, feat), jnp.float32),
        scratch_shapes=[
            pltpu.VMEM((n, feat + 1), jnp.bfloat16),
            pltpu.VMEM((cm, n), jnp.float32),
            pltpu.VMEM((cm, n), jnp.float32),
            pltpu.SemaphoreType.DMA,
            pltpu.SemaphoreType.DMA,
        ],
        compiler_params=pltpu.CompilerParams(
            dimension_semantics=("arbitrary",),
            vmem_limit_bytes=64 * 1024 * 1024,
        ),
    )(adj, xa0, wpack)
